# baseline (speedup 1.0000x reference)
import jax
import jax.numpy as jnp
from jax.experimental import pallas as pl
from jax.experimental.pallas import tpu as pltpu

_H = 64
_P = 8
_T = 2048


def _mlp_body(h_ref, gid_ref, mask_ref, w1_ref, b1_ref, w2a_ref, s_ref, o_ref):
    T = h_ref.shape[0]
    GH = w1_ref.shape[1]
    G = GH // _H

    x = h_ref[...].astype(jnp.bfloat16)
    h1 = jnp.dot(x, w1_ref[...], preferred_element_type=jnp.float32) + b1_ref[...]
    h1 = jnp.maximum(h1, 0.0)

    gid = gid_ref[...]
    h1c = jnp.zeros((T, _H), jnp.float32)
    for g in range(G):
        h1c = h1c + jnp.where(gid == g, h1[:, g * _H : (g + 1) * _H], 0.0)

    lane128 = jax.lax.broadcasted_iota(jnp.int32, (T, 2 * _H), 1)
    aug = jnp.where(lane128 == _H, 1.0, 0.0)
    h1a = (jnp.pad(h1c, ((0, 0), (0, _H))) + aug).astype(jnp.bfloat16)

    la = jnp.dot(h1a, w2a_ref[...], preferred_element_type=jnp.float32)
    la_m = jnp.where(lane128 // _P == gid, la, 0.0).astype(jnp.bfloat16)
    sel = jnp.dot(la_m, s_ref[...], preferred_element_type=jnp.float32)

    logits = jnp.where(mask_ref[...] > 0, sel, -1e9)
    m = jnp.max(logits, axis=1, keepdims=True)
    e = jnp.exp(logits - m)
    o_ref[...] = e / jnp.sum(e, axis=1, keepdims=True)


def kernel(h_int, group_ids, feasible_mask, W1, b1, W2, b2):
    B, D = h_int.shape
    G, _, H = W1.shape
    P = W2.shape[2]
    GP = G * P

    W1r = W1.transpose(1, 0, 2).reshape(D, G * H).astype(jnp.bfloat16)
    b1r = b1.reshape(1, G * H)
    W2cat = W2.transpose(1, 0, 2).reshape(H, GP)
    W2aug = jnp.concatenate(
        [W2cat, b2.reshape(1, GP), jnp.zeros((H - 1, GP), jnp.float32)], axis=0
    ).astype(jnp.bfloat16)
    S = (jnp.arange(GP)[:, None] % P == jnp.arange(P)[None, :]).astype(jnp.bfloat16)
    gid2 = group_ids.reshape(B, 1)
    maskf = feasible_mask.astype(jnp.float32)

    out = pl.pallas_call(
        _mlp_body,
        grid=(B // _T,),
        in_specs=[
            pl.BlockSpec((_T, D), lambda i: (i, 0)),
            pl.BlockSpec((_T, 1), lambda i: (i, 0)),
            pl.BlockSpec((_T, P), lambda i: (i, 0)),
            pl.BlockSpec((D, G * H), lambda i: (0, 0)),
            pl.BlockSpec((1, G * H), lambda i: (0, 0)),
            pl.BlockSpec((2 * H, GP), lambda i: (0, 0)),
            pl.BlockSpec((GP, P), lambda i: (0, 0)),
        ],
        out_specs=pl.BlockSpec((_T, P), lambda i: (i, 0)),
        out_shape=jax.ShapeDtypeStruct((B, P), jnp.float32),
    )(h_int, gid2, maskf, W1r, b1r, W2aug, S)
    return out


# speedup vs baseline: 1.2545x; 1.2545x over previous
import jax
import jax.numpy as jnp
from jax.experimental import pallas as pl
from jax.experimental.pallas import tpu as pltpu

_H = 64
_P = 8
_T = 2048


def _mlp_body(h_ref, gid_ref, mask_ref, w1_ref, b1_ref, w2a_ref, s_ref,
              lg_ref, lp_ref, o_ref):
    T = h_ref.shape[0]
    GH = w1_ref.shape[1]
    G = GH // _H

    x = h_ref[...].astype(jnp.bfloat16)
    h1 = jnp.dot(x, w1_ref[...], preferred_element_type=jnp.float32) + b1_ref[...]
    h1 = jnp.maximum(h1, 0.0)

    gid = gid_ref[...]
    h1m = jnp.where(lg_ref[...] == gid, h1, 0.0)
    h1c = jnp.zeros((T, _H), jnp.float32)
    for g in range(G):
        h1c = h1c + h1m[:, g * _H : (g + 1) * _H]

    lane128 = jax.lax.broadcasted_iota(jnp.int32, (T, 2 * _H), 1)
    aug = jnp.where(lane128 == _H, 1.0, 0.0)
    h1a = (jnp.pad(h1c, ((0, 0), (0, _H))) + aug).astype(jnp.bfloat16)

    la = jnp.dot(h1a, w2a_ref[...], preferred_element_type=jnp.float32)
    la_m = jnp.where(lp_ref[...] == gid, la, 0.0).astype(jnp.bfloat16)
    sel = jnp.dot(la_m, s_ref[...], preferred_element_type=jnp.float32)

    logits = jnp.where(mask_ref[...] > 0, sel, -1e9)
    m = jnp.max(logits, axis=1, keepdims=True)
    e = jnp.exp(logits - m)
    o_ref[...] = e / jnp.sum(e, axis=1, keepdims=True)


def kernel(h_int, group_ids, feasible_mask, W1, b1, W2, b2):
    B, D = h_int.shape
    G, _, H = W1.shape
    P = W2.shape[2]
    GP = G * P

    W1r = W1.transpose(1, 0, 2).reshape(D, G * H).astype(jnp.bfloat16)
    b1r = b1.reshape(1, G * H)
    W2cat = W2.transpose(1, 0, 2).reshape(H, GP)
    W2aug = jnp.concatenate(
        [W2cat, b2.reshape(1, GP), jnp.zeros((H - 1, GP), jnp.float32)], axis=0
    ).astype(jnp.bfloat16)
    S = (jnp.arange(GP)[:, None] % P == jnp.arange(P)[None, :]).astype(jnp.bfloat16)
    gid2 = group_ids.reshape(B, 1)
    maskf = feasible_mask.astype(jnp.float32)
    lane_g = (jnp.arange(G * H, dtype=jnp.int32) // H).reshape(1, G * H)
    lane_p = (jnp.arange(GP, dtype=jnp.int32) // P).reshape(1, GP)

    out = pl.pallas_call(
        _mlp_body,
        grid=(B // _T,),
        in_specs=[
            pl.BlockSpec((_T, D), lambda i: (i, 0)),
            pl.BlockSpec((_T, 1), lambda i: (i, 0)),
            pl.BlockSpec((_T, P), lambda i: (i, 0)),
            pl.BlockSpec((D, G * H), lambda i: (0, 0)),
            pl.BlockSpec((1, G * H), lambda i: (0, 0)),
            pl.BlockSpec((2 * H, GP), lambda i: (0, 0)),
            pl.BlockSpec((GP, P), lambda i: (0, 0)),
            pl.BlockSpec((1, G * H), lambda i: (0, 0)),
            pl.BlockSpec((1, GP), lambda i: (0, 0)),
        ],
        out_specs=pl.BlockSpec((_T, P), lambda i: (i, 0)),
        out_shape=jax.ShapeDtypeStruct((B, P), jnp.float32),
    )(h_int, gid2, maskf, W1r, b1r, W2aug, S, lane_g, lane_p)
    return out


# speedup vs baseline: 1.2677x; 1.0105x over previous
import jax
import jax.numpy as jnp
from jax.experimental import pallas as pl
from jax.experimental.pallas import tpu as pltpu

_H = 64
_P = 8
_T = 2048


def _mlp_body(h_ref, gid_ref, mask_ref, w1_ref, b1_ref, w2a_ref, s_ref, o_ref):
    T = h_ref.shape[0]
    GH = w1_ref.shape[1]
    G = GH // _H

    x = h_ref[...].astype(jnp.bfloat16)
    h1 = jnp.dot(x, w1_ref[...], preferred_element_type=jnp.float32) + b1_ref[...]
    h1 = jnp.maximum(h1, 0.0)

    gid = gid_ref[...]
    lane_g = jax.lax.broadcasted_iota(jnp.int32, (T, GH), 1) // _H
    h1m = jnp.where(lane_g == gid, h1, 0.0)
    h1c = jnp.zeros((T, _H), jnp.float32)
    for g in range(G):
        h1c = h1c + h1m[:, g * _H : (g + 1) * _H]

    lane128 = jax.lax.broadcasted_iota(jnp.int32, (T, 2 * _H), 1)
    aug = jnp.where(lane128 == _H, 1.0, 0.0)
    h1a = (jnp.pad(h1c, ((0, 0), (0, _H))) + aug).astype(jnp.bfloat16)

    la = jnp.dot(h1a, w2a_ref[...], preferred_element_type=jnp.float32)
    la_m = jnp.where(lane128 // _P == gid, la, 0.0).astype(jnp.bfloat16)
    sel = jnp.dot(la_m, s_ref[...], preferred_element_type=jnp.float32)

    logits = jnp.where(mask_ref[...] > 0, sel, -1e9)
    m = jnp.max(logits, axis=1, keepdims=True)
    e = jnp.exp(logits - m)
    o_ref[...] = e / jnp.sum(e, axis=1, keepdims=True)


def kernel(h_int, group_ids, feasible_mask, W1, b1, W2, b2):
    B, D = h_int.shape
    G, _, H = W1.shape
    P = W2.shape[2]
    GP = G * P

    W1r = W1.transpose(1, 0, 2).reshape(D, G * H).astype(jnp.bfloat16)
    b1r = b1.reshape(1, G * H)
    W2cat = W2.transpose(1, 0, 2).reshape(H, GP)
    W2aug = jnp.concatenate(
        [W2cat, b2.reshape(1, GP), jnp.zeros((H - 1, GP), jnp.float32)], axis=0
    ).astype(jnp.bfloat16)
    S = (jnp.arange(GP)[:, None] % P == jnp.arange(P)[None, :]).astype(jnp.bfloat16)
    gid2 = group_ids.reshape(B, 1)
    maskf = feasible_mask.astype(jnp.float32)

    out = pl.pallas_call(
        _mlp_body,
        grid=(B // _T,),
        in_specs=[
            pl.BlockSpec((_T, D), lambda i: (i, 0)),
            pl.BlockSpec((_T, 1), lambda i: (i, 0)),
            pl.BlockSpec((_T, P), lambda i: (i, 0)),
            pl.BlockSpec((D, G * H), lambda i: (0, 0)),
            pl.BlockSpec((1, G * H), lambda i: (0, 0)),
            pl.BlockSpec((2 * H, GP), lambda i: (0, 0)),
            pl.BlockSpec((GP, P), lambda i: (0, 0)),
        ],
        out_specs=pl.BlockSpec((_T, P), lambda i: (i, 0)),
        out_shape=jax.ShapeDtypeStruct((B, P), jnp.float32),
    )(h_int, gid2, maskf, W1r, b1r, W2aug, S)
    return out
